# baseline (speedup 1.0000x reference)
import functools

import jax
import jax.numpy as jnp
from jax.experimental import pallas as pl
from jax.experimental.pallas import tpu as pltpu
from jax.experimental.pallas import tpu_sc as plsc

_EMB = 32
_W = 128
_BM = 2048
_LINEAR_IN = 832
_NPAD = 128


_PB = 2048


def _tp_body(xt_ref, eye_ref, out_ref):
    x = xt_ref[...]
    for r in range(4):
        out_ref[:, r * _EMB:(r + 1) * _EMB] = jax.lax.dot_general(
            x[:, r * _PB:(r + 1) * _PB], eye_ref[...],
            dimension_numbers=(((0,), (0,)), ((), ())),
            preferred_element_type=jnp.float32,
            precision=jax.lax.Precision.HIGHEST,
        )


def _tc_relayout(table, eye):
    v = table.shape[0]
    xt = table.T
    nblk = pl.cdiv(v, 4 * _PB)
    return pl.pallas_call(
        _tp_body,
        grid=(nblk,),
        in_specs=[
            pl.BlockSpec((_EMB, 4 * _PB), lambda i: (0, i)),
            pl.BlockSpec((_EMB, _EMB), lambda i: (0, 0)),
        ],
        out_specs=pl.BlockSpec((_PB, 4 * _EMB), lambda i: (i, 0)),
        out_shape=jax.ShapeDtypeStruct((nblk * _PB, 4 * _EMB), jnp.float32),
    )(xt, eye)


def _sc_gather(table, idx):
    n = idx.shape[1]
    mesh = plsc.VectorSubcoreMesh(core_axis_name="core", subcore_axis_name="subcore")

    @functools.partial(
        pl.kernel,
        out_type=jax.ShapeDtypeStruct((n, _EMB), jnp.float32),
        mesh=mesh,
        compiler_params=pltpu.CompilerParams(use_tc_tiling_on_sc=False),
    )
    def gather_kernel(t_hbm, i_hbm, o_hbm):
        def body(i_vmem, o_vmem):
            pltpu.sync_copy(t_hbm.at[i_vmem.at[0]], o_vmem)

        pltpu.emit_pipeline(
            body,
            grid=(n // _W,),
            in_specs=[pl.BlockSpec((1, _W), index_map=lambda i: (0, i))],
            out_specs=[pl.BlockSpec((_W, _EMB), index_map=lambda i: (i, 0))],
            core_axis_name=("core", "subcore"),
            dimension_semantics=(pltpu.PARALLEL,),
        )(i_hbm, o_hbm)

    return gather_kernel(table, idx)


def _mlp_body(x0_ref, x1_ref, x2_ref, whT_ref, bh_ref, woT_ref, bo_ref, out_ref):
    h = jnp.dot(x0_ref[...], whT_ref[0:640, :], preferred_element_type=jnp.float32)
    h = h + jnp.dot(x1_ref[...], whT_ref[640:800, :], preferred_element_type=jnp.float32)
    h = h + jnp.dot(x2_ref[...], whT_ref[800:832, :], preferred_element_type=jnp.float32)
    h = jnp.maximum(h + bh_ref[...], 0.0)
    out_ref[...] = jnp.dot(h, woT_ref[...], preferred_element_type=jnp.float32) + bo_ref[...]


def _tc_mlp(x0, x1, x2, whT, bh, woT, bo):
    b = x0.shape[0]
    grid = (b // _BM,)
    return pl.pallas_call(
        _mlp_body,
        grid=grid,
        in_specs=[
            pl.BlockSpec((_BM, 20 * _EMB), lambda i: (i, 0)),
            pl.BlockSpec((_BM, 5 * _EMB), lambda i: (i, 0)),
            pl.BlockSpec((_BM, 1 * _EMB), lambda i: (i, 0)),
            pl.BlockSpec((_LINEAR_IN, _NPAD), lambda i: (0, 0)),
            pl.BlockSpec((1, _NPAD), lambda i: (0, 0)),
            pl.BlockSpec((_NPAD, _NPAD), lambda i: (0, 0)),
            pl.BlockSpec((1, _NPAD), lambda i: (0, 0)),
        ],
        out_specs=pl.BlockSpec((_BM, _NPAD), lambda i: (i, 0)),
        out_shape=jax.ShapeDtypeStruct((b, _NPAD), jnp.float32),
    )(x0, x1, x2, whT, bh, woT, bo)


def kernel(features, table0, table1, table2, Wh, bh, Wo, bo):
    b = features.shape[0]
    v = table0.shape[0]
    q = v // 4
    feats = features.astype(jnp.int32)
    w4 = 4 * _PB
    fmap = w4 * (feats // w4) + 4 * (feats % _PB) + (feats % w4) // _PB
    idx0 = fmap[:, 0:20].reshape(1, -1)
    idx1 = fmap[:, 20:25].reshape(1, -1)
    idx2 = fmap[:, 25:26].reshape(1, -1)

    eye = jnp.eye(_EMB, dtype=jnp.float32)
    vp = pl.cdiv(v, w4) * w4
    t0 = _tc_relayout(table0, eye).reshape(vp, _EMB)
    g0 = _sc_gather(t0, idx0)
    t1 = _tc_relayout(table1, eye).reshape(vp, _EMB)
    g1 = _sc_gather(t1, idx1)
    t2 = _tc_relayout(table2, eye).reshape(vp, _EMB)
    g2 = _sc_gather(t2, idx2)

    x0 = g0.reshape(b, 20 * _EMB)
    x1 = g1.reshape(b, 5 * _EMB)
    x2 = g2.reshape(b, 1 * _EMB)

    hid = Wh.shape[0]
    whT = jnp.pad(Wh.T, ((0, 0), (0, _NPAD - hid)))
    bhp = jnp.pad(bh, (0, _NPAD - hid)).reshape(1, _NPAD)
    woT = jnp.pad(Wo.T, ((0, _NPAD - hid), (0, _NPAD - Wo.shape[0])))
    bop = jnp.pad(bo, (0, _NPAD - Wo.shape[0])).reshape(1, _NPAD)

    out = _tc_mlp(x0, x1, x2, whT, bhp, woT, bop)
    return out[:, : Wo.shape[0]].reshape(b, 1, Wo.shape[0])


# speedup vs baseline: 1.8978x; 1.8978x over previous
import functools

import jax
import jax.numpy as jnp
from jax.experimental import pallas as pl
from jax.experimental.pallas import tpu as pltpu
from jax.experimental.pallas import tpu_sc as plsc

_EMB = 32
_W = 128
_BM = 2048
_LINEAR_IN = 832
_NPAD = 128


_PB = 2048


def _tp_body(xt_ref, eye_ref, out_ref):
    x = xt_ref[...]
    for r in range(4):
        out_ref[:, r * _EMB:(r + 1) * _EMB] = jax.lax.dot_general(
            x[:, r * _PB:(r + 1) * _PB], eye_ref[...],
            dimension_numbers=(((0,), (0,)), ((), ())),
            preferred_element_type=jnp.float32,
            precision=jax.lax.Precision.DEFAULT,
        )


def _tc_relayout(table, eye):
    v = table.shape[0]
    xt = table.T
    nblk = pl.cdiv(v, 4 * _PB)
    return pl.pallas_call(
        _tp_body,
        grid=(nblk,),
        in_specs=[
            pl.BlockSpec((_EMB, 4 * _PB), lambda i: (0, i)),
            pl.BlockSpec((_EMB, _EMB), lambda i: (0, 0)),
        ],
        out_specs=pl.BlockSpec((_PB, 4 * _EMB), lambda i: (i, 0)),
        out_shape=jax.ShapeDtypeStruct((nblk * _PB, 4 * _EMB), jnp.float32),
    )(xt, eye)


def _sc_gather(table, idx):
    n = idx.shape[1]
    mesh = plsc.VectorSubcoreMesh(core_axis_name="core", subcore_axis_name="subcore")

    @functools.partial(
        pl.kernel,
        out_type=jax.ShapeDtypeStruct((n, _EMB), jnp.float32),
        mesh=mesh,
        compiler_params=pltpu.CompilerParams(use_tc_tiling_on_sc=False),
    )
    def gather_kernel(t_hbm, i_hbm, o_hbm):
        def body(i_vmem, o_vmem):
            pltpu.sync_copy(t_hbm.at[i_vmem.at[0]], o_vmem)

        pltpu.emit_pipeline(
            body,
            grid=(n // _W,),
            in_specs=[pl.BlockSpec((1, _W), index_map=lambda i: (0, i))],
            out_specs=[pl.BlockSpec((_W, _EMB), index_map=lambda i: (i, 0))],
            core_axis_name=("core", "subcore"),
            dimension_semantics=(pltpu.PARALLEL,),
        )(i_hbm, o_hbm)

    return gather_kernel(table, idx)


def _mlp_body(x0_ref, x1_ref, x2_ref, whT_ref, bh_ref, woT_ref, bo_ref, out_ref):
    h = jnp.dot(x0_ref[...], whT_ref[0:640, :], preferred_element_type=jnp.float32)
    h = h + jnp.dot(x1_ref[...], whT_ref[640:800, :], preferred_element_type=jnp.float32)
    h = h + jnp.dot(x2_ref[...], whT_ref[800:832, :], preferred_element_type=jnp.float32)
    h = jnp.maximum(h + bh_ref[...], 0.0)
    out_ref[...] = jnp.dot(h, woT_ref[...], preferred_element_type=jnp.float32) + bo_ref[...]


def _tc_mlp(x0, x1, x2, whT, bh, woT, bo):
    b = x0.shape[0]
    grid = (b // _BM,)
    return pl.pallas_call(
        _mlp_body,
        grid=grid,
        in_specs=[
            pl.BlockSpec((_BM, 20 * _EMB), lambda i: (i, 0)),
            pl.BlockSpec((_BM, 5 * _EMB), lambda i: (i, 0)),
            pl.BlockSpec((_BM, 1 * _EMB), lambda i: (i, 0)),
            pl.BlockSpec((_LINEAR_IN, _NPAD), lambda i: (0, 0)),
            pl.BlockSpec((1, _NPAD), lambda i: (0, 0)),
            pl.BlockSpec((_NPAD, _NPAD), lambda i: (0, 0)),
            pl.BlockSpec((1, _NPAD), lambda i: (0, 0)),
        ],
        out_specs=pl.BlockSpec((_BM, _NPAD), lambda i: (i, 0)),
        out_shape=jax.ShapeDtypeStruct((b, _NPAD), jnp.float32),
    )(x0, x1, x2, whT, bh, woT, bo)


def kernel(features, table0, table1, table2, Wh, bh, Wo, bo):
    b = features.shape[0]
    v = table0.shape[0]
    q = v // 4
    feats = features.astype(jnp.int32)
    w4 = 4 * _PB
    fmap = w4 * (feats // w4) + 4 * (feats % _PB) + (feats % w4) // _PB
    idx0 = fmap[:, 0:20].reshape(1, -1)
    idx1 = fmap[:, 20:25].reshape(1, -1)
    idx2 = fmap[:, 25:26].reshape(1, -1)

    eye = jnp.eye(_EMB, dtype=jnp.float32)
    vp = pl.cdiv(v, w4) * w4
    t0 = _tc_relayout(table0, eye).reshape(vp, _EMB)
    g0 = _sc_gather(t0, idx0)
    t1 = _tc_relayout(table1, eye).reshape(vp, _EMB)
    g1 = _sc_gather(t1, idx1)
    t2 = _tc_relayout(table2, eye).reshape(vp, _EMB)
    g2 = _sc_gather(t2, idx2)

    x0 = g0.reshape(b, 20 * _EMB)
    x1 = g1.reshape(b, 5 * _EMB)
    x2 = g2.reshape(b, 1 * _EMB)

    hid = Wh.shape[0]
    whT = jnp.pad(Wh.T, ((0, 0), (0, _NPAD - hid)))
    bhp = jnp.pad(bh, (0, _NPAD - hid)).reshape(1, _NPAD)
    woT = jnp.pad(Wo.T, ((0, _NPAD - hid), (0, _NPAD - Wo.shape[0])))
    bop = jnp.pad(bo, (0, _NPAD - Wo.shape[0])).reshape(1, _NPAD)

    out = _tc_mlp(x0, x1, x2, whT, bhp, woT, bop)
    return out[:, : Wo.shape[0]].reshape(b, 1, Wo.shape[0])
